# Initial kernel scaffold; baseline (speedup 1.0000x reference)
import jax
import jax.numpy as jnp
from jax.experimental import pallas as pl


def kernel(x, weights):
    raise NotImplementedError("write your pallas kernel here")


# speedup vs baseline: 1.4372x; 1.4372x over previous
import functools

import jax
import jax.numpy as jnp
from jax import lax
from jax.experimental import pallas as pl
from jax.experimental.pallas import tpu as pltpu
from jax.experimental.pallas import tpu_sc as plsc

_CHUNK = 128


@functools.cache
def _build(n_workers, n_chunks, d, nc):
  mesh = plsc.VectorSubcoreMesh(core_axis_name="c", subcore_axis_name="s")

  @functools.partial(
      pl.kernel,
      out_type=jax.ShapeDtypeStruct((n_workers, n_chunks, _CHUNK, d),
                                    jnp.float32),
      mesh=mesh,
      scratch_types=[
          pltpu.VMEM((n_chunks, _CHUNK), jnp.int32),
          pltpu.VMEM((_CHUNK, d), jnp.float32),
          pltpu.SemaphoreType.DMA,
      ],
      compiler_params=pltpu.CompilerParams(use_tc_tiling_on_sc=False),
  )
  def gather_kernel(table_hbm, idx_hbm, out_hbm, idx_v, rows_v, sem):
    wid = lax.axis_index("s") * nc + lax.axis_index("c")
    pltpu.sync_copy(idx_hbm.at[wid], idx_v)

    def body(j, carry):
      pltpu.async_copy(table_hbm.at[idx_v.at[j]], rows_v, sem).wait()
      pltpu.sync_copy(rows_v, out_hbm.at[wid, j])
      return carry

    lax.fori_loop(0, n_chunks, body, 0)

  return gather_kernel


def kernel(x, weights):
  b, f = x.shape
  v, d = weights.shape
  n = b * f
  mesh = plsc.VectorSubcoreMesh(core_axis_name="c", subcore_axis_name="s")
  nw = mesh.num_cores * mesh.num_subcores
  idx = x.reshape(n).astype(jnp.int32)
  pad = (-n) % (nw * _CHUNK)
  if pad:
    idx = jnp.concatenate([idx, jnp.zeros((pad,), jnp.int32)])
  n_chunks = (n + pad) // (nw * _CHUNK)
  idx = idx.reshape(nw, n_chunks, _CHUNK)
  out = _build(nw, n_chunks, d, mesh.num_cores)(weights, idx)
  return out.reshape((n + pad), d)[:n].reshape(b, f, d)


# speedup vs baseline: 1.5784x; 1.0982x over previous
import functools

import jax
import jax.numpy as jnp
from jax import lax
from jax.experimental import pallas as pl
from jax.experimental.pallas import tpu as pltpu
from jax.experimental.pallas import tpu_sc as plsc

_CHUNK = 128


_NBUF = 8


@functools.cache
def _build(n_workers, n_chunks, d, nc):
  assert n_chunks % _NBUF == 0 and n_chunks // _NBUF >= 2
  n_outer = n_chunks // _NBUF
  mesh = plsc.VectorSubcoreMesh(core_axis_name="c", subcore_axis_name="s")

  @functools.partial(
      pl.kernel,
      out_type=jax.ShapeDtypeStruct((n_workers, n_chunks, _CHUNK, d),
                                    jnp.float32),
      mesh=mesh,
      scratch_types=[
          pltpu.VMEM((n_chunks, _CHUNK), jnp.int32),
          pltpu.VMEM((_NBUF, _CHUNK, d), jnp.float32),
          pltpu.SemaphoreType.DMA((_NBUF,)),
      ],
      compiler_params=pltpu.CompilerParams(use_tc_tiling_on_sc=False),
  )
  def gather_kernel(table_hbm, idx_hbm, out_hbm, idx_v, rows_v, gsem):
    wid = lax.axis_index("s") * nc + lax.axis_index("c")
    pltpu.sync_copy(idx_hbm.at[wid], idx_v)

    for b in range(_NBUF):
      pltpu.async_copy(table_hbm.at[idx_v.at[b]], rows_v.at[b], gsem.at[b])

    def body(o, carry):
      for b in range(_NBUF):
        j = o * _NBUF + b
        pltpu.make_async_copy(table_hbm.at[idx_v.at[j]], rows_v.at[b],
                              gsem.at[b]).wait()
        pltpu.sync_copy(rows_v.at[b], out_hbm.at[wid, j])
        pltpu.async_copy(table_hbm.at[idx_v.at[j + _NBUF]], rows_v.at[b],
                         gsem.at[b])
      return carry

    lax.fori_loop(0, n_outer - 1, body, 0)

    for b in range(_NBUF):
      j = (n_outer - 1) * _NBUF + b
      pltpu.make_async_copy(table_hbm.at[idx_v.at[j]], rows_v.at[b],
                            gsem.at[b]).wait()
      pltpu.sync_copy(rows_v.at[b], out_hbm.at[wid, j])

  return gather_kernel


def kernel(x, weights):
  b, f = x.shape
  v, d = weights.shape
  n = b * f
  mesh = plsc.VectorSubcoreMesh(core_axis_name="c", subcore_axis_name="s")
  nw = mesh.num_cores * mesh.num_subcores
  idx = x.reshape(n).astype(jnp.int32)
  pad = (-n) % (nw * _CHUNK)
  if pad:
    idx = jnp.concatenate([idx, jnp.zeros((pad,), jnp.int32)])
  n_chunks = (n + pad) // (nw * _CHUNK)
  idx = idx.reshape(nw, n_chunks, _CHUNK)
  out = _build(nw, n_chunks, d, mesh.num_cores)(weights, idx)
  return out.reshape((n + pad), d)[:n].reshape(b, f, d)
